# Initial kernel scaffold; baseline (speedup 1.0000x reference)
import jax
import jax.numpy as jnp
from jax.experimental import pallas as pl


def kernel(features, edge_index, W1, b1, W2, b2):
    raise NotImplementedError("write your pallas kernel here")


# speedup vs baseline: 3.8302x; 3.8302x over previous
import functools

import jax
import jax.numpy as jnp
from jax import lax
from jax.experimental import pallas as pl
from jax.experimental.pallas import tpu as pltpu
from jax.experimental.pallas import tpu_sc as plsc

_NC = 2
_NS = 16
_LANES = 16


def _sc_mesh():
    return plsc.VectorSubcoreMesh(
        core_axis_name="c", subcore_axis_name="s",
        num_cores=_NC, num_subcores=_NS)


def _make_deg_kernel(NPAD, D, K, CH):
    RP = NPAD // _NS

    @functools.partial(
        pl.kernel,
        out_type=jax.ShapeDtypeStruct((_NC, NPAD, D), jnp.float32),
        mesh=_sc_mesh(),
        scratch_types=[
            pltpu.VMEM((2 * K, CH), jnp.int32),
            pltpu.VMEM((CH, D), jnp.float32),
            pltpu.VMEM_SHARED((NPAD, D), jnp.float32),
        ])
    def deg_kernel(src_hbm, dst_hbm, ones_hbm, zeros_hbm, out_hbm,
                   idx_v, ones_v, hist):
        c = lax.axis_index("c")
        s = lax.axis_index("s")
        pltpu.sync_copy(zeros_hbm, hist.at[pl.ds(s * RP, RP)])
        pltpu.sync_copy(ones_hbm, ones_v)

        @pl.when(c == 0)
        def _():
            pltpu.sync_copy(src_hbm.at[s], idx_v.at[pl.ds(0, K)])
            pltpu.sync_copy(src_hbm.at[s + _NS], idx_v.at[pl.ds(K, K)])

        @pl.when(c == 1)
        def _():
            pltpu.sync_copy(dst_hbm.at[s], idx_v.at[pl.ds(0, K)])
            pltpu.sync_copy(dst_hbm.at[s + _NS], idx_v.at[pl.ds(K, K)])

        plsc.subcore_barrier()

        def body(k, carry):
            pltpu.sync_copy(ones_v, hist.at[idx_v.at[k]], add=True)
            return carry

        lax.fori_loop(0, 2 * K, body, 0)
        plsc.subcore_barrier()
        pltpu.sync_copy(hist.at[pl.ds(s * RP, RP)],
                        out_hbm.at[c, pl.ds(s * RP, RP)])

    return deg_kernel


def _make_agg_kernel(NPAD, D, K, CH):
    RP = NPAD // _NS

    @functools.partial(
        pl.kernel,
        out_type=jax.ShapeDtypeStruct((_NC, NPAD, D), jnp.float32),
        mesh=_sc_mesh(),
        scratch_types=[
            pltpu.VMEM((K, CH), jnp.int32),
            pltpu.VMEM((K, CH), jnp.int32),
            pltpu.VMEM((CH, D), jnp.float32),
            pltpu.VMEM_SHARED((NPAD, D), jnp.float32),
            pltpu.SemaphoreType.DMA,
        ])
    def agg_kernel(y_hbm, src_hbm, dst_hbm, zeros_hbm, out_hbm,
                   src_v, dst_v, rows_v, z_sh, sem):
        c = lax.axis_index("c")
        s = lax.axis_index("s")
        wid = s * _NC + c
        pltpu.sync_copy(zeros_hbm, z_sh.at[pl.ds(s * RP, RP)])
        pltpu.sync_copy(src_hbm.at[wid], src_v)
        pltpu.sync_copy(dst_hbm.at[wid], dst_v)
        plsc.subcore_barrier()

        def body(k, carry):
            pltpu.async_copy(y_hbm.at[src_v.at[k]], rows_v, sem).wait()
            pltpu.sync_copy(rows_v, z_sh.at[dst_v.at[k]], add=True)
            return carry

        lax.fori_loop(0, K, body, 0)
        plsc.subcore_barrier()
        pltpu.sync_copy(z_sh.at[pl.ds(s * RP, RP)],
                        out_hbm.at[c, pl.ds(s * RP, RP)])

    return agg_kernel


def _mm_first(x, w, do, blk):
    NPAD, D = x.shape

    def body(x_ref, w_ref, do_ref, o_ref):
        scale = lax.rsqrt(jnp.clip(do_ref[...], 1.0, None))
        o_ref[...] = jnp.dot(x_ref[...], w_ref[...],
                             preferred_element_type=jnp.float32) * scale

    return pl.pallas_call(
        body,
        grid=(NPAD // blk,),
        in_specs=[pl.BlockSpec((blk, D), lambda i: (i, 0)),
                  pl.BlockSpec((D, D), lambda i: (0, 0)),
                  pl.BlockSpec((blk, 1), lambda i: (i, 0))],
        out_specs=pl.BlockSpec((blk, D), lambda i: (i, 0)),
        out_shape=jax.ShapeDtypeStruct((NPAD, D), jnp.float32),
    )(x, w, do)


def _mm_mid(z0, z1, di, do, b, w, blk):
    NPAD, D = z0.shape

    def body(z0_ref, z1_ref, di_ref, do_ref, b_ref, w_ref, o_ref):
        si = lax.rsqrt(jnp.clip(di_ref[...], 1.0, None))
        so = lax.rsqrt(jnp.clip(do_ref[...], 1.0, None))
        h = jnp.maximum((z0_ref[...] + z1_ref[...]) * si + b_ref[...], 0.0)
        o_ref[...] = jnp.dot(h, w_ref[...],
                             preferred_element_type=jnp.float32) * so

    dspec = pl.BlockSpec((blk, 1), lambda i: (i, 0))
    return pl.pallas_call(
        body,
        grid=(NPAD // blk,),
        in_specs=[pl.BlockSpec((blk, D), lambda i: (i, 0)),
                  pl.BlockSpec((blk, D), lambda i: (i, 0)),
                  dspec, dspec,
                  pl.BlockSpec((1, D), lambda i: (0, 0)),
                  pl.BlockSpec((D, D), lambda i: (0, 0))],
        out_specs=pl.BlockSpec((blk, D), lambda i: (i, 0)),
        out_shape=jax.ShapeDtypeStruct((NPAD, D), jnp.float32),
    )(z0, z1, di, do, b, w)


def _mm_last(z0, z1, di, b, blk):
    NPAD, D = z0.shape

    def body(z0_ref, z1_ref, di_ref, b_ref, o_ref):
        si = lax.rsqrt(jnp.clip(di_ref[...], 1.0, None))
        o_ref[...] = jnp.maximum(
            (z0_ref[...] + z1_ref[...]) * si + b_ref[...], 0.0)

    return pl.pallas_call(
        body,
        grid=(NPAD // blk,),
        in_specs=[pl.BlockSpec((blk, D), lambda i: (i, 0)),
                  pl.BlockSpec((blk, D), lambda i: (i, 0)),
                  pl.BlockSpec((blk, 1), lambda i: (i, 0)),
                  pl.BlockSpec((1, D), lambda i: (0, 0))],
        out_specs=pl.BlockSpec((blk, D), lambda i: (i, 0)),
        out_shape=jax.ShapeDtypeStruct((NPAD, D), jnp.float32),
    )(z0, z1, di, b)


def kernel(features, edge_index, W1, b1, W2, b2):
    N, D = features.shape
    E = edge_index.shape[1]
    NW = _NC * _NS
    CH = 128
    K = -(-E // (NW * CH))
    EPAD = NW * K * CH
    BLK = 1024
    NPAD = (N // BLK + 1) * BLK

    feat_pad = jnp.pad(features, ((0, NPAD - N), (0, 0)))
    sent = jnp.full((EPAD - E,), NPAD - 1, dtype=jnp.int32)
    src_r = jnp.concatenate([edge_index[0], sent]).reshape(NW, K, CH)
    dst_r = jnp.concatenate([edge_index[1], sent]).reshape(NW, K, CH)
    onesD = jnp.ones((CH, D), jnp.float32)
    zerosD = jnp.zeros((NPAD // _NS, D), jnp.float32)
    b1_2d = b1.reshape(1, D)
    b2_2d = b2.reshape(1, D)

    deg_kernel = _make_deg_kernel(NPAD, D, K, CH)
    agg_kernel = _make_agg_kernel(NPAD, D, K, CH)

    deg_h = deg_kernel(src_r, dst_r, onesD, zerosD)
    do = deg_h[0, :, 0:1]
    di = deg_h[1, :, 0:1]

    y1 = _mm_first(feat_pad, W1, do, BLK)
    z1p = agg_kernel(y1, src_r, dst_r, zerosD)
    y2 = _mm_mid(z1p[0], z1p[1], di, do, b1_2d, W2, BLK)
    z2p = agg_kernel(y2, src_r, dst_r, zerosD)
    out = _mm_last(z2p[0], z2p[1], di, b2_2d, BLK)
    return out[:N]


# speedup vs baseline: 3.8315x; 1.0003x over previous
import functools

import jax
import jax.numpy as jnp
from jax import lax
from jax.experimental import pallas as pl
from jax.experimental.pallas import tpu as pltpu
from jax.experimental.pallas import tpu_sc as plsc

_NC = 2
_NS = 16


def _sc_mesh():
    return plsc.VectorSubcoreMesh(
        core_axis_name="c", subcore_axis_name="s",
        num_cores=_NC, num_subcores=_NS)


def _make_deg_kernel(NPAD, D, K, CH):
    RP = NPAD // _NS

    @functools.partial(
        pl.kernel,
        out_type=jax.ShapeDtypeStruct((_NC, NPAD, D), jnp.float32),
        mesh=_sc_mesh(),
        scratch_types=[
            pltpu.VMEM((2 * K, CH), jnp.int32),
            pltpu.VMEM((CH, D), jnp.float32),
            pltpu.VMEM_SHARED((NPAD, D), jnp.float32),
        ])
    def deg_kernel(src_hbm, dst_hbm, ones_hbm, zeros_hbm, out_hbm,
                   idx_v, ones_v, hist):
        c = lax.axis_index("c")
        s = lax.axis_index("s")
        pltpu.sync_copy(zeros_hbm, hist.at[pl.ds(s * RP, RP)])
        pltpu.sync_copy(ones_hbm, ones_v)

        @pl.when(c == 0)
        def _():
            pltpu.sync_copy(src_hbm.at[s], idx_v.at[pl.ds(0, K)])
            pltpu.sync_copy(src_hbm.at[s + _NS], idx_v.at[pl.ds(K, K)])

        @pl.when(c == 1)
        def _():
            pltpu.sync_copy(dst_hbm.at[s], idx_v.at[pl.ds(0, K)])
            pltpu.sync_copy(dst_hbm.at[s + _NS], idx_v.at[pl.ds(K, K)])

        plsc.subcore_barrier()

        def body(k, carry):
            pltpu.sync_copy(ones_v, hist.at[idx_v.at[k]], add=True)
            return carry

        lax.fori_loop(0, 2 * K, body, 0)
        plsc.subcore_barrier()
        pltpu.sync_copy(hist.at[pl.ds(s * RP, RP)],
                        out_hbm.at[c, pl.ds(s * RP, RP)])

    return deg_kernel


def _make_agg_kernel(NPAD, D, K, CH):
    RP = NPAD // _NS

    @functools.partial(
        pl.kernel,
        out_type=jax.ShapeDtypeStruct((_NC, NPAD, D), jnp.float32),
        mesh=_sc_mesh(),
        scratch_types=[
            pltpu.VMEM((K, CH), jnp.int32),
            pltpu.VMEM((K, CH), jnp.int32),
            pltpu.VMEM((CH, D), jnp.float32),
            pltpu.VMEM_SHARED((NPAD, D), jnp.float32),
            pltpu.SemaphoreType.DMA,
        ])
    def agg_kernel(y_hbm, src_hbm, dst_hbm, zeros_hbm, out_hbm,
                   src_v, dst_v, rows_v, z_sh, sem):
        c = lax.axis_index("c")
        s = lax.axis_index("s")
        wid = s * _NC + c
        pltpu.sync_copy(zeros_hbm, z_sh.at[pl.ds(s * RP, RP)])
        pltpu.sync_copy(src_hbm.at[wid], src_v)
        pltpu.sync_copy(dst_hbm.at[wid], dst_v)
        plsc.subcore_barrier()

        def body(k, carry):
            pltpu.async_copy(y_hbm.at[src_v.at[k]], rows_v, sem).wait()
            pltpu.sync_copy(rows_v, z_sh.at[dst_v.at[k]], add=True)
            return carry

        lax.fori_loop(0, K, body, 0)
        plsc.subcore_barrier()
        pltpu.sync_copy(z_sh.at[pl.ds(s * RP, RP)],
                        out_hbm.at[c, pl.ds(s * RP, RP)])

    return agg_kernel


def _mm_first(x, w, do, blk):
    NPAD, D = x.shape

    def body(x_ref, w_ref, do_ref, o_ref):
        scale = lax.rsqrt(jnp.clip(do_ref[...], 1.0, None))
        o_ref[...] = jnp.dot(x_ref[...], w_ref[...],
                             preferred_element_type=jnp.float32) * scale

    return pl.pallas_call(
        body,
        grid=(NPAD // blk,),
        in_specs=[pl.BlockSpec((blk, D), lambda i: (i, 0)),
                  pl.BlockSpec((D, D), lambda i: (0, 0)),
                  pl.BlockSpec((blk, 1), lambda i: (i, 0))],
        out_specs=pl.BlockSpec((blk, D), lambda i: (i, 0)),
        out_shape=jax.ShapeDtypeStruct((NPAD, D), jnp.float32),
    )(x, w, do)


def _mm_mid(z0, z1, di, do, b, w, blk):
    NPAD, D = z0.shape

    def body(z0_ref, z1_ref, di_ref, do_ref, b_ref, w_ref, o_ref):
        si = lax.rsqrt(jnp.clip(di_ref[...], 1.0, None))
        so = lax.rsqrt(jnp.clip(do_ref[...], 1.0, None))
        h = jnp.maximum((z0_ref[...] + z1_ref[...]) * si + b_ref[...], 0.0)
        o_ref[...] = jnp.dot(h, w_ref[...],
                             preferred_element_type=jnp.float32) * so

    dspec = pl.BlockSpec((blk, 1), lambda i: (i, 0))
    return pl.pallas_call(
        body,
        grid=(NPAD // blk,),
        in_specs=[pl.BlockSpec((blk, D), lambda i: (i, 0)),
                  pl.BlockSpec((blk, D), lambda i: (i, 0)),
                  dspec, dspec,
                  pl.BlockSpec((1, D), lambda i: (0, 0)),
                  pl.BlockSpec((D, D), lambda i: (0, 0))],
        out_specs=pl.BlockSpec((blk, D), lambda i: (i, 0)),
        out_shape=jax.ShapeDtypeStruct((NPAD, D), jnp.float32),
    )(z0, z1, di, do, b, w)


def _mm_last(z0, z1, di, b, blk):
    NPAD, D = z0.shape

    def body(z0_ref, z1_ref, di_ref, b_ref, o_ref):
        si = lax.rsqrt(jnp.clip(di_ref[...], 1.0, None))
        o_ref[...] = jnp.maximum(
            (z0_ref[...] + z1_ref[...]) * si + b_ref[...], 0.0)

    return pl.pallas_call(
        body,
        grid=(NPAD // blk,),
        in_specs=[pl.BlockSpec((blk, D), lambda i: (i, 0)),
                  pl.BlockSpec((blk, D), lambda i: (i, 0)),
                  pl.BlockSpec((blk, 1), lambda i: (i, 0)),
                  pl.BlockSpec((1, D), lambda i: (0, 0))],
        out_specs=pl.BlockSpec((blk, D), lambda i: (i, 0)),
        out_shape=jax.ShapeDtypeStruct((NPAD, D), jnp.float32),
    )(z0, z1, di, b)


def kernel(features, edge_index, W1, b1, W2, b2):
    N, D = features.shape
    E = edge_index.shape[1]
    NW = _NC * _NS
    CH = 128
    K = -(-E // (NW * CH))
    EPAD = NW * K * CH
    BLK = 1024
    NPAD = (N // BLK + 1) * BLK

    feat_pad = jnp.pad(features, ((0, NPAD - N), (0, 0)))
    sent = jnp.full((EPAD - E,), NPAD - 1, dtype=jnp.int32)
    src_r = jnp.concatenate([edge_index[0], sent]).reshape(NW, K, CH)
    dst_r = jnp.concatenate([edge_index[1], sent]).reshape(NW, K, CH)
    onesD = jnp.ones((CH, D), jnp.float32)
    zerosD = jnp.zeros((NPAD // _NS, D), jnp.float32)
    b1_2d = b1.reshape(1, D)
    b2_2d = b2.reshape(1, D)

    deg_kernel = _make_deg_kernel(NPAD, D, K, CH)
    agg_kernel = _make_agg_kernel(NPAD, D, K, CH)

    deg_h = deg_kernel(src_r, dst_r, onesD, zerosD)
    do = deg_h[0, :, 0:1]
    di = deg_h[1, :, 0:1]

    y1 = _mm_first(feat_pad, W1, do, BLK)
    z1p = agg_kernel(y1, src_r, dst_r, zerosD)
    y2 = _mm_mid(z1p[0], z1p[1], di, do, b1_2d, W2, BLK)
    z2p = agg_kernel(y2, src_r, dst_r, zerosD)
    out = _mm_last(z2p[0], z2p[1], di, b2_2d, BLK)
    return out[:N]
